# baseline (speedup 1.0000x reference)
import functools

import jax
import jax.numpy as jnp
from jax import lax
from jax.experimental import pallas as pl
from jax.experimental.pallas import tpu as pltpu
from jax.experimental.pallas import tpu_sc as plsc

CH = 128
LAT = 64
NUM_GRAPHS = 64
MAX_NODES = 64
OUT_SIZE = MAX_NODES * (MAX_NODES - 1) // 2

NC = 2
NS = 16
KE = 125
KP = 128
STRIPE = 640


def _sc_degree_kernel(E, NACC):
    nrows = E // KE
    cpt = nrows // (NC * NS)
    rows_h = NACC // 128
    rpt = rows_h // NS
    WAVE = 10
    mesh = plsc.VectorSubcoreMesh(core_axis_name="c", subcore_axis_name="s")

    @functools.partial(
        pl.kernel, mesh=mesh,
        out_type=jax.ShapeDtypeStruct((NC, rows_h, 128), jnp.float32),
        scratch_types=[
            pltpu.VMEM((cpt, KP), jnp.int32),
            pltpu.VMEM((KP,), jnp.float32),
            pltpu.VMEM((128,), jnp.float32),
            pltpu.VMEM_SHARED((NACC,), jnp.float32),
            pltpu.SemaphoreType.DMA,
        ],
    )
    def deg_kernel(dstp_hbm, out_hbm, di_v, ones_v, tmp_v, acc_sh, sem):
        c = lax.axis_index("c")
        s = lax.axis_index("s")
        w = c * NS + s
        pltpu.sync_copy(dstp_hbm.at[pl.ds(w * cpt, cpt)], di_v)
        for j in range(KP // 16):
            ones_v[pl.ds(j * 16, 16)] = jnp.full((16,), 1.0, jnp.float32)
        for j in range(8):
            tmp_v[pl.ds(j * 16, 16)] = jnp.zeros((16,), jnp.float32)
        for t in range(rpt):
            pltpu.sync_copy(tmp_v, acc_sh.at[pl.ds((s * rpt + t) * 128, 128)])
        plsc.subcore_barrier()

        def body(i, carry):
            for u in range(WAVE):
                pltpu.async_copy(ones_v, acc_sh.at[di_v.at[i * WAVE + u]],
                                 sem, add=True)
            for u in range(WAVE):
                pltpu.make_async_copy(ones_v, acc_sh.at[di_v.at[0]], sem).wait()
            return carry

        lax.fori_loop(0, cpt // WAVE, body, 0)
        plsc.subcore_barrier()
        for t in range(rpt):
            row = s * rpt + t
            pltpu.sync_copy(acc_sh.at[pl.ds(row * 128, 128)], tmp_v)
            pltpu.sync_copy(tmp_v, out_hbm.at[c, row])

    return deg_kernel


def _sc_scatter_kernel(E, N, NACC, C):
    nrows = E // KE
    cpt = nrows // (NC * NS)
    half = cpt // 2
    mesh = plsc.VectorSubcoreMesh(core_axis_name="c", subcore_axis_name="s")

    @functools.partial(
        pl.kernel, mesh=mesh,
        out_type=jax.ShapeDtypeStruct((NC, N, C), jnp.float32),
        scratch_types=[
            pltpu.VMEM((half, KP), jnp.int32),
            pltpu.VMEM((half, KP), jnp.int32),
            pltpu.VMEM((KP, CH), jnp.float32),
            pltpu.VMEM((KP, CH), jnp.float32),
            pltpu.VMEM_SHARED((NACC, CH), jnp.float32),
            pltpu.SemaphoreType.DMA,
            pltpu.SemaphoreType.DMA,
        ],
    )
    def scat_kernel(rows_hbm, srcp_hbm, dstp_hbm, out_hbm, si_v, di_v,
                    rows0, rows1, acc_sh, sem0, sem1):
        c = lax.axis_index("c")
        s = lax.axis_index("s")
        w = c * NS + s

        def zbody(i, carry):
            for j in range(C // 16):
                rows0[i, pl.ds(j * 16, 16)] = jnp.zeros((16,), jnp.float32)
            return carry

        lax.fori_loop(0, KP, zbody, 0)
        nz = STRIPE // 80
        for t in range(nz):
            start = s * STRIPE + t * 80

            @pl.when(start < NACC)
            def _():
                pltpu.sync_copy(rows0.at[pl.ds(0, 80)],
                                acc_sh.at[pl.ds(start, 80)])

        plsc.subcore_barrier()

        def gather(j, buf, sem):
            pltpu.async_copy(rows_hbm.at[si_v.at[j]], buf, sem)

        def gwait(buf, sem):
            pltpu.make_async_copy(rows_hbm.at[si_v.at[0]], buf, sem).wait()

        def scat(j, buf):
            pltpu.sync_copy(buf, acc_sh.at[di_v.at[j]], add=True)

        for h in range(2):
            base = w * cpt + h * half
            pltpu.sync_copy(srcp_hbm.at[pl.ds(base, half)], si_v)
            pltpu.sync_copy(dstp_hbm.at[pl.ds(base, half)], di_v)
            gather(0, rows0, sem0)

            def body(i, carry):
                j = 2 * i
                gather(j + 1, rows1, sem1)
                gwait(rows0, sem0)
                scat(j, rows0)
                gather(j + 2, rows0, sem0)
                gwait(rows1, sem1)
                scat(j + 1, rows1)
                return carry

            lax.fori_loop(0, (half - 2) // 2, body, 0)
            gather(half - 1, rows1, sem1)
            gwait(rows0, sem0)
            scat(half - 2, rows0)
            gwait(rows1, sem1)
            scat(half - 1, rows1)

        plsc.subcore_barrier()
        for t in range(nz):
            start = s * STRIPE + t * 80

            @pl.when(start < N)
            def _():
                pltpu.sync_copy(acc_sh.at[pl.ds(start, 80)],
                                rows0.at[pl.ds(0, 80)])
                pltpu.sync_copy(rows0.at[pl.ds(0, 80)],
                                out_hbm.at[c, pl.ds(start, 80)])

    return scat_kernel


_R = 1000


def _tc_scale_body(x_ref, w_ref, deg_ref, hp_ref, dinv_ref):
    deg = deg_ref[0] + deg_ref[1] + 1.0
    dinv = lax.rsqrt(jnp.maximum(deg, 1.0))
    h = jnp.dot(x_ref[...], w_ref[...], preferred_element_type=jnp.float32)
    hp_ref[...] = h * dinv
    dinv_ref[...] = dinv


def _tc_scale(x, W1, degp):
    n = x.shape[0]
    grid = (n // _R,)
    return pl.pallas_call(
        _tc_scale_body,
        grid=grid,
        in_specs=[
            pl.BlockSpec((_R, CH), lambda i: (i, 0)),
            pl.BlockSpec((CH, CH), lambda i: (0, 0)),
            pl.BlockSpec((NC, _R, 1), lambda i: (0, i, 0)),
        ],
        out_specs=[
            pl.BlockSpec((_R, CH), lambda i: (i, 0)),
            pl.BlockSpec((_R, 1), lambda i: (i, 0)),
        ],
        out_shape=[
            jax.ShapeDtypeStruct((n, CH), jnp.float32),
            jax.ShapeDtypeStruct((n, 1), jnp.float32),
        ],
    )(x, W1, degp)


def _tc_mid_body(s_ref, hp_ref, dinv_ref, b_ref, w_ref, hp2_ref):
    dinv = dinv_ref[...]
    agg = s_ref[0] + s_ref[1] + hp_ref[...]
    x2 = jax.nn.relu(dinv * agg + b_ref[...])
    h2 = jnp.dot(x2, w_ref[...], preferred_element_type=jnp.float32)
    hp2_ref[...] = h2 * dinv


def _tc_mid(S1, hp1, dinv, b1, W2):
    n = hp1.shape[0]
    grid = (n // _R,)
    return pl.pallas_call(
        _tc_mid_body,
        grid=grid,
        in_specs=[
            pl.BlockSpec((NC, _R, CH), lambda i: (0, i, 0)),
            pl.BlockSpec((_R, CH), lambda i: (i, 0)),
            pl.BlockSpec((_R, 1), lambda i: (i, 0)),
            pl.BlockSpec((1, CH), lambda i: (0, 0)),
            pl.BlockSpec((CH, CH), lambda i: (0, 0)),
        ],
        out_specs=pl.BlockSpec((_R, CH), lambda i: (i, 0)),
        out_shape=jax.ShapeDtypeStruct((n, CH), jnp.float32),
    )(S1, hp1, dinv, b1, W2)


def _tc_pool_body(s_ref, hp_ref, dinv_ref, b_ref, batch_ref, hg_ref,
                  pool_ref, cnt_ref):
    i = pl.program_id(0)

    @pl.when(i == 0)
    def _():
        pool_ref[...] = jnp.zeros_like(pool_ref)
        cnt_ref[...] = jnp.zeros_like(cnt_ref)

    dinv = dinv_ref[...]
    agg = s_ref[0] + s_ref[1] + hp_ref[...]
    h = jax.nn.relu(dinv * agg + b_ref[...])
    gi = lax.broadcasted_iota(jnp.int32, (_R, NUM_GRAPHS), 1)
    ind = jnp.where(gi == batch_ref[...], 1.0, 0.0)
    dn = (((0,), (0,)), ((), ()))
    pool_ref[...] += lax.dot_general(ind, h, dn,
                                     preferred_element_type=jnp.float32)
    cnt_ref[...] += lax.dot_general(ind, jnp.ones((_R, 1), jnp.float32), dn,
                                    preferred_element_type=jnp.float32)

    @pl.when(i == pl.num_programs(0) - 1)
    def _():
        hg_ref[...] = pool_ref[...] / jnp.maximum(cnt_ref[...], 1.0)


def _tc_pool(S2, hp2, dinv, b2, batch2d):
    n = hp2.shape[0]
    grid = (n // _R,)
    return pl.pallas_call(
        _tc_pool_body,
        grid=grid,
        in_specs=[
            pl.BlockSpec((NC, _R, CH), lambda i: (0, i, 0)),
            pl.BlockSpec((_R, CH), lambda i: (i, 0)),
            pl.BlockSpec((_R, 1), lambda i: (i, 0)),
            pl.BlockSpec((1, CH), lambda i: (0, 0)),
            pl.BlockSpec((_R, 1), lambda i: (i, 0)),
        ],
        out_specs=pl.BlockSpec((NUM_GRAPHS, CH), lambda i: (0, 0)),
        out_shape=jax.ShapeDtypeStruct((NUM_GRAPHS, CH), jnp.float32),
        scratch_shapes=[
            pltpu.VMEM((NUM_GRAPHS, CH), jnp.float32),
            pltpu.VMEM((NUM_GRAPHS, 1), jnp.float32),
        ],
    )(S2, hp2, dinv, b2, batch2d)


def _tc_decoder_body(hg_ref, muW_ref, mub_ref, lvW_ref, lvb_ref, eps_ref,
                     d1w_ref, d1b_ref, d2w_ref, d2b_ref, d3w_ref, d3b_ref,
                     adj_ref, mu_ref, lv_ref):
    hg = hg_ref[...]
    mu = jnp.dot(hg, muW_ref[...], preferred_element_type=jnp.float32) + mub_ref[...]
    lv = jnp.dot(hg, lvW_ref[...], preferred_element_type=jnp.float32) + lvb_ref[...]
    mu_ref[...] = mu
    lv_ref[...] = lv
    z = mu + eps_ref[...] * jnp.exp(0.5 * lv)
    p = jax.nn.relu(jnp.dot(z, d1w_ref[...], preferred_element_type=jnp.float32) + d1b_ref[...])
    p = jax.nn.relu(jnp.dot(p, d2w_ref[...], preferred_element_type=jnp.float32) + d2b_ref[...])
    logits = jnp.dot(p, d3w_ref[...], preferred_element_type=jnp.float32) + d3b_ref[...]
    probs = jax.nn.sigmoid(logits)
    adj_ref[...] = jnp.zeros((NUM_GRAPHS, MAX_NODES, MAX_NODES), jnp.float32)
    off = 0
    for r in range(MAX_NODES - 1):
        w = MAX_NODES - 1 - r
        adj_ref[:, r, pl.ds(r + 1, w)] = probs[:, off:off + w]
        off += w
    a = adj_ref[...]
    adj_ref[...] = a + jnp.swapaxes(a, 1, 2)


def _tc_decoder(hg, mu_W, mu_b, lv_W, lv_b, eps, D1_W, D1_b, D2_W, D2_b,
                D3_W, D3_b):
    return pl.pallas_call(
        _tc_decoder_body,
        out_shape=[
            jax.ShapeDtypeStruct((NUM_GRAPHS, MAX_NODES, MAX_NODES), jnp.float32),
            jax.ShapeDtypeStruct((NUM_GRAPHS, LAT), jnp.float32),
            jax.ShapeDtypeStruct((NUM_GRAPHS, LAT), jnp.float32),
        ],
    )(hg, mu_W, mu_b.reshape(1, LAT), lv_W, lv_b.reshape(1, LAT), eps,
      D1_W, D1_b.reshape(1, CH), D2_W, D2_b.reshape(1, CH),
      D3_W, D3_b.reshape(1, OUT_SIZE))


def kernel(x, edge_index, batch, W1, b1, W2, b2, mu_W, mu_b, lv_W, lv_b,
           D1_W, D1_b, D2_W, D2_b, D3_W, D3_b):
    n, c = x.shape
    e = edge_index.shape[1]
    src = edge_index[0].astype(jnp.int32)
    dst = edge_index[1].astype(jnp.int32)
    batch2d = batch.astype(jnp.int32).reshape(n, 1)

    nrows = e // KE
    nacc = n + 80
    ndeg = -(-(nacc) // 2048) * 2048
    trash = 80
    npadw = KP - KE
    ri = jnp.arange(nrows, dtype=jnp.int32)[:, None]
    ci = jnp.arange(npadw, dtype=jnp.int32)[None, :]
    pad_dst = n + (ri * npadw + ci) % trash
    src_p = jnp.concatenate(
        [src.reshape(nrows, KE), jnp.zeros((nrows, npadw), jnp.int32)], axis=1)
    dst_p = jnp.concatenate([dst.reshape(nrows, KE), pad_dst], axis=1)

    degp = _sc_degree_kernel(e, ndeg)(dst_p)
    degp3 = degp.reshape(NC, ndeg, 1)[:, :n, :]
    hp1, dinv = _tc_scale(x, W1, degp3)
    S1 = _sc_scatter_kernel(e, n, nacc, c)(hp1, src_p, dst_p)
    hp2 = _tc_mid(S1, hp1, dinv, b1.reshape(1, CH), W2)
    S2 = _sc_scatter_kernel(e, n, nacc, c)(hp2, src_p, dst_p)
    hg = _tc_pool(S2, hp2, dinv, b2.reshape(1, CH), batch2d)
    eps = jax.random.normal(jax.random.key(42), (NUM_GRAPHS, LAT), jnp.float32)
    adj, mu, lv = _tc_decoder(hg, mu_W, mu_b, lv_W, lv_b, eps,
                              D1_W, D1_b, D2_W, D2_b, D3_W, D3_b)
    return adj, mu, lv
